# baseline (speedup 1.0000x reference)
import jax
import jax.numpy as jnp
from jax.experimental import pallas as pl


def _bf(ref):
    return ref[...].astype(jnp.bfloat16)


def _mlp_kernel(x_ref, w1_ref, b1_ref, w2_ref, b2_ref,
                wa1_ref, ba1_ref, wa2_ref, ba2_ref,
                wc1_ref, bc1_ref, wc2_ref, bc2_ref,
                logits_ref, value_ref):
    n_act = wa2_ref.shape[1]
    xt = _bf(x_ref)
    h = jnp.maximum(jax.lax.dot_general(
        xt, _bf(w1_ref), (((0,), (0,)), ((), ())),
        preferred_element_type=jnp.float32) + b1_ref[...], 0.0
    ).astype(jnp.bfloat16)
    e = jnp.maximum(
        jnp.dot(h, _bf(w2_ref), preferred_element_type=jnp.float32)
        + b2_ref[...], 0.0).astype(jnp.bfloat16)
    wh1 = jnp.concatenate([_bf(wa1_ref), _bf(wc1_ref)], axis=1)
    bh1 = jnp.concatenate([ba1_ref[...], bc1_ref[...]], axis=1)
    ac = jnp.maximum(
        jnp.dot(e, wh1, preferred_element_type=jnp.float32)
        + bh1, 0.0).astype(jnp.bfloat16)
    half = wa1_ref.shape[0]
    wh2 = jnp.concatenate([
        jnp.concatenate([_bf(wa2_ref), jnp.zeros((half, 1), jnp.bfloat16)], axis=1),
        jnp.concatenate([jnp.zeros((half, n_act), jnp.bfloat16), _bf(wc2_ref)],
                        axis=1),
    ], axis=0)
    out_t = jax.lax.dot_general(
        wh2, ac, (((0,), (1,)), ((), ())),
        preferred_element_type=jnp.float32)
    bh2 = jnp.concatenate([ba2_ref[...], bc2_ref[...]], axis=1)
    out_t = out_t + jax.lax.dot_general(
        bh2, jnp.ones((1, 1), jnp.float32), (((0,), (0,)), ((), ())),
        preferred_element_type=jnp.float32)
    logits_ref[...] = out_t[:n_act, :]
    value_ref[...] = out_t[n_act:n_act + 1, :]


def kernel(global_state, W1, b1, W2, b2, Wa1, ba1, Wa2, ba2, Wc1, bc1, Wc2, bc2):
    B, in_dim = global_state.shape
    n_act = Wa2.shape[1]

    def whole(a):
        return pl.BlockSpec(a.shape, lambda: (0,) * a.ndim)

    b1r, b2r = b1[None, :], b2[None, :]
    ba1r, ba2r = ba1[None, :], ba2[None, :]
    bc1r, bc2r = bc1[None, :], bc2[None, :]

    xt = global_state.T
    logits, value = pl.pallas_call(
        _mlp_kernel,
        in_specs=[
            whole(xt),
            whole(W1), whole(b1r), whole(W2), whole(b2r),
            whole(Wa1), whole(ba1r), whole(Wa2), whole(ba2r),
            whole(Wc1), whole(bc1r), whole(Wc2), whole(bc2r),
        ],
        out_specs=[
            pl.BlockSpec((n_act, B), lambda: (0, 0)),
            pl.BlockSpec((1, B), lambda: (0, 0)),
        ],
        out_shape=[
            jax.ShapeDtypeStruct((n_act, B), jnp.float32),
            jax.ShapeDtypeStruct((1, B), jnp.float32),
        ],
    )(xt, W1, b1r, W2, b2r, Wa1, ba1r, Wa2, ba2r, Wc1, bc1r, Wc2, bc2r)
    return (logits.T, value.reshape(B, 1))


# speedup vs baseline: 1.0050x; 1.0050x over previous
import jax
import jax.numpy as jnp
from jax.experimental import pallas as pl


def _bf(ref):
    return ref[...].astype(jnp.bfloat16)


def _mlp_kernel(x_ref, w1_ref, b1_ref, w2_ref, b2_ref,
                wa1_ref, ba1_ref, wa2_ref, ba2_ref,
                wc1_ref, bc1_ref, wc2_ref, bc2_ref,
                logits_ref, value_ref):
    n_act = wa2_ref.shape[1]
    xt = x_ref[...]
    h = jnp.maximum(jax.lax.dot_general(
        xt, w1_ref[...], (((0,), (0,)), ((), ())),
        preferred_element_type=jnp.float32) + b1_ref[...], 0.0)
    e = jnp.maximum(
        jnp.dot(h, w2_ref[...], preferred_element_type=jnp.float32)
        + b2_ref[...], 0.0)
    wh1 = jnp.concatenate([wa1_ref[...], wc1_ref[...]], axis=1)
    bh1 = jnp.concatenate([ba1_ref[...], bc1_ref[...]], axis=1)
    ac = jnp.maximum(
        jnp.dot(e, wh1, preferred_element_type=jnp.float32)
        + bh1, 0.0)
    half = wa1_ref.shape[0]
    wh2 = jnp.concatenate([
        jnp.concatenate([wa2_ref[...], jnp.zeros((half, 1), jnp.float32)], axis=1),
        jnp.concatenate([jnp.zeros((half, n_act), jnp.float32), wc2_ref[...]],
                        axis=1),
    ], axis=0)
    out_t = jax.lax.dot_general(
        wh2, ac, (((0,), (1,)), ((), ())),
        preferred_element_type=jnp.float32)
    bh2 = jnp.concatenate([ba2_ref[...], bc2_ref[...]], axis=1)
    out_t = out_t + jax.lax.dot_general(
        bh2, jnp.ones((1, 1), jnp.float32), (((0,), (0,)), ((), ())),
        preferred_element_type=jnp.float32)
    logits_ref[...] = out_t[:n_act, :]
    value_ref[...] = out_t[n_act:n_act + 1, :]


def kernel(global_state, W1, b1, W2, b2, Wa1, ba1, Wa2, ba2, Wc1, bc1, Wc2, bc2):
    B, in_dim = global_state.shape
    n_act = Wa2.shape[1]

    def whole(a):
        return pl.BlockSpec(a.shape, lambda: (0,) * a.ndim)

    b1r, b2r = b1[None, :], b2[None, :]
    ba1r, ba2r = ba1[None, :], ba2[None, :]
    bc1r, bc2r = bc1[None, :], bc2[None, :]

    xt = global_state.T
    logits, value = pl.pallas_call(
        _mlp_kernel,
        in_specs=[
            whole(xt),
            whole(W1), whole(b1r), whole(W2), whole(b2r),
            whole(Wa1), whole(ba1r), whole(Wa2), whole(ba2r),
            whole(Wc1), whole(bc1r), whole(Wc2), whole(bc2r),
        ],
        out_specs=[
            pl.BlockSpec((n_act, B), lambda: (0, 0)),
            pl.BlockSpec((1, B), lambda: (0, 0)),
        ],
        out_shape=[
            jax.ShapeDtypeStruct((n_act, B), jnp.float32),
            jax.ShapeDtypeStruct((1, B), jnp.float32),
        ],
    )(xt, W1, b1r, W2, b2r, Wa1, ba1r, Wa2, ba2r, Wc1, bc1r, Wc2, bc2r)
    return (logits.T, value.reshape(B, 1))
